# Initial kernel scaffold; baseline (speedup 1.0000x reference)
import jax
import jax.numpy as jnp
from jax.experimental import pallas as pl


def kernel(slots, eW1, eb1, eW2, eb2, eln_g, eln_b, eW3, eb3, nW1, nb1, nW2, nb2, nln_g, nln_b, nW3, nb3, mW, mb):
    raise NotImplementedError("write your pallas kernel here")


# speedup vs baseline: 20.1159x; 20.1159x over previous
import jax
import jax.numpy as jnp
from jax.experimental import pallas as pl

_B, _N, _D, _H = 512, 32, 128, 64
_BB = 8


def _layernorm(x, g, b, eps=1e-5):
    mu = jnp.mean(x, axis=-1, keepdims=True)
    var = jnp.mean((x - mu) ** 2, axis=-1, keepdims=True)
    return (x - mu) * jax.lax.rsqrt(var + eps) * g + b


def _block_kernel(slots_ref, eW1_ref, eb1_ref, eW2_ref, eb2_ref, eln_g_ref,
                  eln_b_ref, eW3_ref, eb3_ref, nW1_ref, nb1_ref, nW2_ref,
                  nb2_ref, nln_g_ref, nln_b_ref, nW3_ref, nb3_ref, mW_ref,
                  mb_ref, out_ref):
    X = slots_ref[...].reshape(_BB * _N, _D)

    eW1 = eW1_ref[...]
    A = jnp.dot(X, eW1[:_D, :], preferred_element_type=jnp.float32)
    Bv = jnp.dot(X, eW1[_D:, :], preferred_element_type=jnp.float32)
    A = A.reshape(_BB, _N, 1, _H)
    Bv = Bv.reshape(_BB, 1, _N, _H)
    h = jax.nn.relu(A + Bv + eb1_ref[...])
    h = h.reshape(_BB * _N * _N, _H)

    h = jnp.dot(h, eW2_ref[...], preferred_element_type=jnp.float32) + eb2_ref[...]
    h = jax.nn.relu(_layernorm(h, eln_g_ref[...], eln_b_ref[...]))
    e = jnp.dot(h, eW3_ref[...], preferred_element_type=jnp.float32) + eb3_ref[...]

    e = e.reshape(_BB, _N, _N, _H)
    ii = jax.lax.broadcasted_iota(jnp.int32, (_BB, _N, _N, _H), 1)
    jj = jax.lax.broadcasted_iota(jnp.int32, (_BB, _N, _N, _H), 2)
    agg = jnp.sum(jnp.where(ii != jj, e, 0.0), axis=2).reshape(_BB * _N, _H)

    nW1 = nW1_ref[...]
    u = (jnp.dot(X, nW1[:_D, :], preferred_element_type=jnp.float32)
         + jnp.dot(agg, nW1[_D:, :], preferred_element_type=jnp.float32)
         + nb1_ref[...])
    u = jax.nn.relu(u)
    u = jnp.dot(u, nW2_ref[...], preferred_element_type=jnp.float32) + nb2_ref[...]
    u = jax.nn.relu(_layernorm(u, nln_g_ref[...], nln_b_ref[...]))
    node_out = jnp.dot(u, nW3_ref[...], preferred_element_type=jnp.float32) + nb3_ref[...]
    node_out = jax.nn.relu(node_out)

    pooled = jnp.sum(node_out.reshape(_BB, _N, _D), axis=1)
    out_ref[...] = (jnp.dot(pooled, mW_ref[...], preferred_element_type=jnp.float32)
                    + mb_ref[...])


def kernel(slots, eW1, eb1, eW2, eb2, eln_g, eln_b, eW3, eb3,
           nW1, nb1, nW2, nb2, nln_g, nln_b, nW3, nb3, mW, mb):
    grid = (_B // _BB,)

    def _full(a):
        return pl.BlockSpec(a.shape, lambda i: (0,) * a.ndim)

    weights = (eW1, eb1, eW2, eb2, eln_g, eln_b, eW3, eb3,
               nW1, nb1, nW2, nb2, nln_g, nln_b, nW3, nb3, mW, mb)
    in_specs = [pl.BlockSpec((_BB, _N, _D), lambda i: (i, 0, 0))]
    in_specs += [_full(w) for w in weights]

    return pl.pallas_call(
        _block_kernel,
        grid=grid,
        in_specs=in_specs,
        out_specs=pl.BlockSpec((_BB, 2 * 8), lambda i: (i, 0)),
        out_shape=jax.ShapeDtypeStruct((_B, 2 * 8), jnp.float32),
    )(slots, *weights)


# speedup vs baseline: 21.4781x; 1.0677x over previous
import jax
import jax.numpy as jnp
from jax.experimental import pallas as pl

_B, _N, _D, _H = 512, 32, 128, 64
_BB = 32


def _layernorm(x, g, b, eps=1e-5):
    mu = jnp.mean(x, axis=-1, keepdims=True)
    var = jnp.mean((x - mu) ** 2, axis=-1, keepdims=True)
    return (x - mu) * jax.lax.rsqrt(var + eps) * g + b


def _block_kernel(slots_ref, eW1_ref, eb1_ref, eW2_ref, eb2_ref, eln_g_ref,
                  eln_b_ref, eW3_ref, eb3_ref, nW1_ref, nb1_ref, nW2_ref,
                  nb2_ref, nln_g_ref, nln_b_ref, nW3_ref, nb3_ref, mW_ref,
                  mb_ref, out_ref):
    X = slots_ref[...].reshape(_BB * _N, _D)

    eW1 = eW1_ref[...]
    A = jnp.dot(X, eW1[:_D, :], preferred_element_type=jnp.float32)
    Bv = jnp.dot(X, eW1[_D:, :], preferred_element_type=jnp.float32)
    A = A.reshape(_BB, _N, 1, _H)
    Bv = Bv.reshape(_BB, 1, _N, _H)
    h = jax.nn.relu(A + Bv + eb1_ref[...])
    h = h.reshape(_BB * _N * _N, _H)

    h = jnp.dot(h, eW2_ref[...], preferred_element_type=jnp.float32) + eb2_ref[...]
    h = jax.nn.relu(_layernorm(h, eln_g_ref[...], eln_b_ref[...]))
    e = jnp.dot(h, eW3_ref[...], preferred_element_type=jnp.float32) + eb3_ref[...]

    e = e.reshape(_BB, _N, _N, _H)
    ii = jax.lax.broadcasted_iota(jnp.int32, (_BB, _N, _N, _H), 1)
    jj = jax.lax.broadcasted_iota(jnp.int32, (_BB, _N, _N, _H), 2)
    agg = jnp.sum(jnp.where(ii != jj, e, 0.0), axis=2).reshape(_BB * _N, _H)

    nW1 = nW1_ref[...]
    u = (jnp.dot(X, nW1[:_D, :], preferred_element_type=jnp.float32)
         + jnp.dot(agg, nW1[_D:, :], preferred_element_type=jnp.float32)
         + nb1_ref[...])
    u = jax.nn.relu(u)
    u = jnp.dot(u, nW2_ref[...], preferred_element_type=jnp.float32) + nb2_ref[...]
    u = jax.nn.relu(_layernorm(u, nln_g_ref[...], nln_b_ref[...]))
    node_out = jnp.dot(u, nW3_ref[...], preferred_element_type=jnp.float32) + nb3_ref[...]
    node_out = jax.nn.relu(node_out)

    pooled = jnp.sum(node_out.reshape(_BB, _N, _D), axis=1)
    out_ref[...] = (jnp.dot(pooled, mW_ref[...], preferred_element_type=jnp.float32)
                    + mb_ref[...])


def kernel(slots, eW1, eb1, eW2, eb2, eln_g, eln_b, eW3, eb3,
           nW1, nb1, nW2, nb2, nln_g, nln_b, nW3, nb3, mW, mb):
    grid = (_B // _BB,)

    def _full(a):
        return pl.BlockSpec(a.shape, lambda i: (0,) * a.ndim)

    weights = (eW1, eb1, eW2, eb2, eln_g, eln_b, eW3, eb3,
               nW1, nb1, nW2, nb2, nln_g, nln_b, nW3, nb3, mW, mb)
    in_specs = [pl.BlockSpec((_BB, _N, _D), lambda i: (i, 0, 0))]
    in_specs += [_full(w) for w in weights]

    return pl.pallas_call(
        _block_kernel,
        grid=grid,
        in_specs=in_specs,
        out_specs=pl.BlockSpec((_BB, 2 * 8), lambda i: (i, 0)),
        out_shape=jax.ShapeDtypeStruct((_B, 2 * 8), jnp.float32),
    )(slots, *weights)


# speedup vs baseline: 31.3411x; 1.4592x over previous
import jax
import jax.numpy as jnp
from jax.experimental import pallas as pl

_B, _N, _D, _H = 512, 32, 128, 64
_BB = 32
_EPS = 1e-5


def _ln_relu(c, J, g, b):
    var = jnp.dot(c * c, J, preferred_element_type=jnp.float32)
    return jax.nn.relu(c * jax.lax.rsqrt(var + _EPS) * g + b)


def _block_kernel(slots_ref, eW1_ref, eb1_ref, eW2c_ref, eb2c_ref, eln_g_ref,
                  eln_b_ref, eW3_ref, eb3_ref, nW1_ref, nb1_ref, nW2c_ref,
                  nb2c_ref, nln_g_ref, nln_b_ref, nW3_ref, nb3_ref, mW_ref,
                  mb_ref, J_ref, out_ref):
    X = slots_ref[...].reshape(_BB * _N, _D)
    J = J_ref[...]

    eW1 = eW1_ref[...]
    A = jnp.dot(X, eW1[:_D, :], preferred_element_type=jnp.float32) + eb1_ref[...]
    Bv = jnp.dot(X, eW1[_D:, :], preferred_element_type=jnp.float32)
    h = jax.nn.relu(A.reshape(_BB, _N, 1, _H) + Bv.reshape(_BB, 1, _N, _H))
    h = h.reshape(_BB * _N * _N, _H)

    c = jnp.dot(h, eW2c_ref[...], preferred_element_type=jnp.float32) + eb2c_ref[...]
    y = _ln_relu(c, J, eln_g_ref[...], eln_b_ref[...])

    hd = jax.nn.relu(A + Bv)
    cd = jnp.dot(hd, eW2c_ref[...], preferred_element_type=jnp.float32) + eb2c_ref[...]
    yd = _ln_relu(cd, J, eln_g_ref[...], eln_b_ref[...])

    ysum = jnp.sum(y.reshape(_BB, _N, _N, _H), axis=2).reshape(_BB * _N, _H)
    agg = (jnp.dot(ysum - yd, eW3_ref[...], preferred_element_type=jnp.float32)
           + 31.0 * eb3_ref[...])

    nW1 = nW1_ref[...]
    u = (jnp.dot(X, nW1[:_D, :], preferred_element_type=jnp.float32)
         + jnp.dot(agg, nW1[_D:, :], preferred_element_type=jnp.float32)
         + nb1_ref[...])
    u = jax.nn.relu(u)
    c2 = jnp.dot(u, nW2c_ref[...], preferred_element_type=jnp.float32) + nb2c_ref[...]
    y2 = _ln_relu(c2, J, nln_g_ref[...], nln_b_ref[...])
    node_out = jnp.dot(y2, nW3_ref[...], preferred_element_type=jnp.float32) + nb3_ref[...]
    node_out = jax.nn.relu(node_out)

    pooled = jnp.sum(node_out.reshape(_BB, _N, _D), axis=1)
    out_ref[...] = (jnp.dot(pooled, mW_ref[...], preferred_element_type=jnp.float32)
                    + mb_ref[...])


def kernel(slots, eW1, eb1, eW2, eb2, eln_g, eln_b, eW3, eb3,
           nW1, nb1, nW2, nb2, nln_g, nln_b, nW3, nb3, mW, mb):
    C = jnp.eye(_H, dtype=jnp.float32) - 1.0 / _H
    eW2c = eW2 @ C
    eb2c = eb2 @ C
    nW2c = nW2 @ C
    nb2c = nb2 @ C
    J = jnp.full((_H, _H), 1.0 / _H, jnp.float32)

    grid = (_B // _BB,)

    def _full(a):
        return pl.BlockSpec(a.shape, lambda i: (0,) * a.ndim)

    weights = (eW1, eb1, eW2c, eb2c, eln_g, eln_b, eW3, eb3,
               nW1, nb1, nW2c, nb2c, nln_g, nln_b, nW3, nb3, mW, mb, J)
    in_specs = [pl.BlockSpec((_BB, _N, _D), lambda i: (i, 0, 0))]
    in_specs += [_full(w) for w in weights]

    return pl.pallas_call(
        _block_kernel,
        grid=grid,
        in_specs=in_specs,
        out_specs=pl.BlockSpec((_BB, 2 * 8), lambda i: (i, 0)),
        out_shape=jax.ShapeDtypeStruct((_B, 2 * 8), jnp.float32),
    )(slots, *weights)


# speedup vs baseline: 41.6526x; 1.3290x over previous
import jax
import jax.numpy as jnp
from jax.experimental import pallas as pl

_B, _N, _D, _H = 512, 32, 128, 64
_BB = 32
_EPS = 1e-5


def _ln_relu(c, J, g, b):
    var = jnp.dot(c * c, J, preferred_element_type=jnp.float32)
    return jax.nn.relu(c * jax.lax.rsqrt(var + _EPS) * g + b)


def _block_kernel(slots_ref, slots2_ref, eW1tD_ref, eb1d_ref, eW1bD_ref,
                  eW1bBD_ref, eW2d_ref, eb2d_ref, eg2_ref, ebb2_ref, Jd_ref,
                  eb3_ref, eW3s_ref, nW1_ref, nb1_ref, nW2c_ref,
                  nb2c_ref, nln_g_ref, nln_b_ref, nW3_ref, nb3_ref, mW_ref,
                  mb_ref, out_ref):
    X = slots_ref[...].reshape(_BB * _N, _D)
    X2 = slots2_ref[...].reshape(_BB * _N // 2, 2 * _D)

    A2 = jnp.dot(X, eW1tD_ref[...], preferred_element_type=jnp.float32) + eb1d_ref[...]
    B2 = jnp.dot(X2, eW1bBD_ref[...], preferred_element_type=jnp.float32)
    h = jax.nn.relu(A2.reshape(_BB, _N, 1, 2 * _H)
                    + B2.reshape(_BB, 1, _N // 2, 2 * _H))
    h = h.reshape(_BB * _N * _N // 2, 2 * _H)

    c = jnp.dot(h, eW2d_ref[...], preferred_element_type=jnp.float32) + eb2d_ref[...]
    y = _ln_relu(c, Jd_ref[...], eg2_ref[...], ebb2_ref[...])

    Bd2 = jnp.dot(X, eW1bD_ref[...], preferred_element_type=jnp.float32)
    hd = jax.nn.relu(A2 + Bd2)
    cd = jnp.dot(hd, eW2d_ref[...], preferred_element_type=jnp.float32) + eb2d_ref[...]
    yd = _ln_relu(cd, Jd_ref[...], eg2_ref[...], ebb2_ref[...])

    ysum = jnp.sum(y.reshape(_BB, _N, _N // 2, 2 * _H), axis=2)
    ysum = ysum.reshape(_BB * _N, 2 * _H)
    agg = (jnp.dot(ysum - 0.5 * yd, eW3s_ref[...],
                   preferred_element_type=jnp.float32)
           + 31.0 * eb3_ref[...])

    nW1 = nW1_ref[...]
    u = (jnp.dot(X, nW1[:_D, :], preferred_element_type=jnp.float32)
         + jnp.dot(agg, nW1[_D:, :], preferred_element_type=jnp.float32)
         + nb1_ref[...])
    u = jax.nn.relu(u)
    c2 = jnp.dot(u, nW2c_ref[...], preferred_element_type=jnp.float32) + nb2c_ref[...]
    Jh = Jd_ref[0:_H, 0:_H]
    y2 = _ln_relu(c2, Jh, nln_g_ref[...], nln_b_ref[...])
    node_out = jnp.dot(y2, nW3_ref[...], preferred_element_type=jnp.float32) + nb3_ref[...]
    node_out = jax.nn.relu(node_out)

    pooled = jnp.sum(node_out.reshape(_BB, _N, _D), axis=1)
    out_ref[...] = (jnp.dot(pooled, mW_ref[...], preferred_element_type=jnp.float32)
                    + mb_ref[...])


def kernel(slots, eW1, eb1, eW2, eb2, eln_g, eln_b, eW3, eb3,
           nW1, nb1, nW2, nb2, nln_g, nln_b, nW3, nb3, mW, mb):
    C = jnp.eye(_H, dtype=jnp.float32) - 1.0 / _H
    eW2c = eW2 @ C
    eb2c = eb2 @ C
    nW2c = nW2 @ C
    nb2c = nb2 @ C
    J = jnp.full((_H, _H), 1.0 / _H, jnp.float32)
    Z = jnp.zeros((_H, _H), jnp.float32)
    ZD = jnp.zeros((_D, _H), jnp.float32)
    eW1t, eW1b = eW1[:_D], eW1[_D:]
    eW1tD = jnp.concatenate([eW1t, eW1t], axis=1)
    eW1bD = jnp.concatenate([eW1b, eW1b], axis=1)
    eW1bBD = jnp.block([[eW1b, ZD], [ZD, eW1b]])
    eb1d = jnp.concatenate([eb1, eb1])
    eW2d = jnp.block([[eW2c, Z], [Z, eW2c]])
    Jd = jnp.block([[J, Z], [Z, J]])
    eb2d = jnp.concatenate([eb2c, eb2c])
    eg2 = jnp.concatenate([eln_g, eln_g])
    ebb2 = jnp.concatenate([eln_b, eln_b])
    eW3s = jnp.concatenate([eW3, eW3], axis=0)
    slots2 = slots.reshape(_B, _N // 2, 2 * _D)

    grid = (_B // _BB,)

    def _full(a):
        return pl.BlockSpec(a.shape, lambda i: (0,) * a.ndim)

    weights = (eW1tD, eb1d, eW1bD, eW1bBD, eW2d, eb2d, eg2, ebb2, Jd,
               eb3, eW3s, nW1, nb1, nW2c, nb2c, nln_g, nln_b,
               nW3, nb3, mW, mb)
    in_specs = [pl.BlockSpec((_BB, _N, _D), lambda i: (i, 0, 0)),
                pl.BlockSpec((_BB, _N // 2, 2 * _D), lambda i: (i, 0, 0))]
    in_specs += [_full(w) for w in weights]

    return pl.pallas_call(
        _block_kernel,
        grid=grid,
        in_specs=in_specs,
        out_specs=pl.BlockSpec((_BB, 2 * 8), lambda i: (i, 0)),
        out_shape=jax.ShapeDtypeStruct((_B, 2 * 8), jnp.float32),
    )(slots, slots2, *weights)
